# baseline (speedup 1.0000x reference)
import functools

import jax
import jax.numpy as jnp
from jax import lax
from jax.experimental import pallas as pl
from jax.experimental.pallas import tpu as pltpu
from jax.experimental.pallas import tpu_sc as plsc

_SOLVE_ITERS = 8
_ROW_BLOCK = 32

_SC_ROWS = 32
_SC_SOLVE_ITERS = 8
_L = 16
_UNROLL = 16


def _sparsemax_block(x_ref, o_ref):
    zmax = jnp.max(x_ref[...], axis=-1, keepdims=True)
    lo = zmax - 1.0
    hi = zmax
    f_lo = jnp.sum(jnp.maximum(x_ref[...] - lo, 0.0), axis=-1,
                   keepdims=True) - 1.0
    t_p = lo - 1.0
    f_p = f_lo + 1.0

    def step(_, carry):
        lo, hi, f_lo, t_p, f_p = carry
        mid = 0.5 * (lo + hi)
        sec = lo + f_lo * (lo - t_p) / jnp.maximum(f_p - f_lo, 1e-30)
        t = jnp.where(sec < hi, jnp.maximum(sec, mid), mid)
        ft = jnp.sum(jnp.maximum(x_ref[...] - t, 0.0), axis=-1,
                     keepdims=True) - 1.0
        ge = ft >= 0.0
        return (
            jnp.where(ge, t, lo),
            jnp.where(ge, hi, t),
            jnp.where(ge, ft, f_lo),
            jnp.where(ge, lo, t_p),
            jnp.where(ge, f_lo, f_p),
        )

    lo, hi, f_lo, t_p, f_p = jax.lax.fori_loop(
        0, _SOLVE_ITERS, step, (lo, hi, f_lo, t_p, f_p))

    cnt = jnp.sum((x_ref[...] > lo).astype(jnp.float32), axis=-1,
                  keepdims=True)
    tau = lo + f_lo / jnp.maximum(cnt, 1.0)
    o_ref[...] = jnp.maximum(x_ref[...] - tau, 0.0)


def _tc_sparsemax(x):
    n_rows, d = x.shape
    grid = (n_rows // _ROW_BLOCK,)
    return pl.pallas_call(
        _sparsemax_block,
        grid=grid,
        in_specs=[pl.BlockSpec((_ROW_BLOCK, d), lambda i: (i, 0))],
        out_specs=pl.BlockSpec((_ROW_BLOCK, d), lambda i: (i, 0)),
        out_shape=jax.ShapeDtypeStruct((n_rows, d), x.dtype),
        compiler_params=pltpu.CompilerParams(
            dimension_semantics=("parallel",),
        ),
    )(x)


def _sc_reduce_pass(row_v, d, acc_op, merge_op, init):
    span = _L * _UNROLL

    def body(i, accs):
        base = i * span
        new = []
        for j in range(_UNROLL):
            v = row_v[pl.ds(base + j * _L, _L)]
            new.append(acc_op(accs[j], v))
        return tuple(new)

    accs = tuple(jnp.full((_L,), init, jnp.float32) for _ in range(_UNROLL))
    accs = lax.fori_loop(0, d // span, body, accs)
    out = accs[0]
    for j in range(1, _UNROLL):
        out = merge_op(out, accs[j])
    return out


def _sc_sparsemax_body(x_hbm, o_hbm, row_v, out_v, sem):
    d = x_hbm.shape[-1]
    wid = lax.axis_index("s") * 2 + lax.axis_index("c")

    pltpu.sync_copy(x_hbm.at[wid], row_v)

    def lane_reduce(vec, op):
        s = vec[0]
        for i in range(1, _L):
            s = op(s, vec[i])
        return jnp.full((_L,), s, jnp.float32)

    def recip(x):
        seed = jnp.int32(0x7EF311C7) - lax.bitcast_convert_type(x, jnp.int32)
        r = lax.bitcast_convert_type(seed, jnp.float32)
        for _ in range(3):
            r = r * (2.0 - x * r)
        return r

    add = lambda a, b: a + b
    mvec = _sc_reduce_pass(row_v, d, jnp.maximum, jnp.maximum, -jnp.inf)
    zmax = lane_reduce(mvec, jnp.maximum)

    def relu_sum(t):
        acc = _sc_reduce_pass(
            row_v, d, lambda a, v: a + jnp.maximum(v - t, 0.0), add, 0.0)
        return lane_reduce(acc, add)

    lo = zmax - 1.0
    hi = zmax
    f_lo = relu_sum(lo) - 1.0
    t_p = lo - 1.0
    f_p = f_lo + 1.0

    def step(_, carry):
        lo, hi, f_lo, t_p, f_p = carry
        mid = 0.5 * (lo + hi)
        sec = lo + f_lo * (lo - t_p) * recip(jnp.maximum(f_p - f_lo, 1e-20))
        t = jnp.where(sec < hi, jnp.maximum(sec, mid), mid)
        ft = relu_sum(t) - 1.0
        ge = ft >= 0.0
        return (
            jnp.where(ge, t, lo),
            jnp.where(ge, hi, t),
            jnp.where(ge, ft, f_lo),
            jnp.where(ge, lo, t_p),
            jnp.where(ge, f_lo, f_p),
        )

    lo, hi, f_lo, t_p, f_p = lax.fori_loop(
        0, _SC_SOLVE_ITERS, step, (lo, hi, f_lo, t_p, f_p))

    cnt = _sc_reduce_pass(
        row_v, d,
        lambda a, v: a + jnp.where(v > lo, 1.0, 0.0), add, 0.0)
    cnt = lane_reduce(cnt, add)
    tau = lo + f_lo * recip(jnp.maximum(cnt, 1.0))

    span = _L * _UNROLL

    def out_body(i, carry):
        base = i * span
        for j in range(_UNROLL):
            sl = pl.ds(base + j * _L, _L)
            out_v[sl] = jnp.maximum(row_v[sl] - tau, 0.0)
        return carry

    lax.fori_loop(0, d // span, out_body, 0)
    pltpu.sync_copy(out_v, o_hbm.at[wid])


def _sc_sparsemax(x):
    n_rows, d = x.shape
    mesh = plsc.VectorSubcoreMesh(core_axis_name="c", subcore_axis_name="s")
    k = functools.partial(
        pl.kernel,
        mesh=mesh,
        out_type=jax.ShapeDtypeStruct((n_rows, d), jnp.float32),
        scratch_types=[
            pltpu.VMEM((d,), jnp.float32),
            pltpu.VMEM((d,), jnp.float32),
            pltpu.SemaphoreType.DMA,
        ],
    )(_sc_sparsemax_body)
    return k(x)


@jax.jit
def kernel(input):
    n_rows, _ = input.shape
    n_tc = n_rows - _SC_ROWS
    out_tc = _tc_sparsemax(input[:n_tc])
    out_sc = _sc_sparsemax(input[n_tc:])
    return jnp.concatenate([out_tc, out_sc], axis=0)


# speedup vs baseline: 1.0063x; 1.0063x over previous
import functools

import jax
import jax.numpy as jnp
from jax import lax
from jax.experimental import pallas as pl
from jax.experimental.pallas import tpu as pltpu
from jax.experimental.pallas import tpu_sc as plsc

_SOLVE_ITERS = 8
_ROW_BLOCK = 32

_SC_ROWS = 32
_SC_SOLVE_ITERS = 6
_L = 16
_UNROLL = 32


def _sparsemax_block(x_ref, o_ref):
    zmax = jnp.max(x_ref[...], axis=-1, keepdims=True)
    lo = zmax - 1.0
    hi = zmax
    f_lo = jnp.sum(jnp.maximum(x_ref[...] - lo, 0.0), axis=-1,
                   keepdims=True) - 1.0
    t_p = lo - 1.0
    f_p = f_lo + 1.0

    def step(_, carry):
        lo, hi, f_lo, t_p, f_p = carry
        mid = 0.5 * (lo + hi)
        sec = lo + f_lo * (lo - t_p) / jnp.maximum(f_p - f_lo, 1e-30)
        t = jnp.where(sec < hi, jnp.maximum(sec, mid), mid)
        ft = jnp.sum(jnp.maximum(x_ref[...] - t, 0.0), axis=-1,
                     keepdims=True) - 1.0
        ge = ft >= 0.0
        return (
            jnp.where(ge, t, lo),
            jnp.where(ge, hi, t),
            jnp.where(ge, ft, f_lo),
            jnp.where(ge, lo, t_p),
            jnp.where(ge, f_lo, f_p),
        )

    lo, hi, f_lo, t_p, f_p = jax.lax.fori_loop(
        0, _SOLVE_ITERS, step, (lo, hi, f_lo, t_p, f_p))

    cnt = jnp.sum((x_ref[...] > lo).astype(jnp.float32), axis=-1,
                  keepdims=True)
    tau = lo + f_lo / jnp.maximum(cnt, 1.0)
    o_ref[...] = jnp.maximum(x_ref[...] - tau, 0.0)


def _tc_sparsemax(x):
    n_rows, d = x.shape
    grid = (n_rows // _ROW_BLOCK,)
    return pl.pallas_call(
        _sparsemax_block,
        grid=grid,
        in_specs=[pl.BlockSpec((_ROW_BLOCK, d), lambda i: (i, 0))],
        out_specs=pl.BlockSpec((_ROW_BLOCK, d), lambda i: (i, 0)),
        out_shape=jax.ShapeDtypeStruct((n_rows, d), x.dtype),
        compiler_params=pltpu.CompilerParams(
            dimension_semantics=("parallel",),
        ),
    )(x)


def _sc_reduce_pass(row_v, d, acc_op, merge_op, init):
    span = _L * _UNROLL

    def body(i, accs):
        base = i * span
        new = []
        for j in range(_UNROLL):
            v = row_v[pl.ds(base + j * _L, _L)]
            new.append(acc_op(accs[j], v))
        return tuple(new)

    accs = tuple(jnp.full((_L,), init, jnp.float32) for _ in range(_UNROLL))
    accs = lax.fori_loop(0, d // span, body, accs)
    out = accs[0]
    for j in range(1, _UNROLL):
        out = merge_op(out, accs[j])
    return out


def _sc_sparsemax_body(x_hbm, o_hbm, row_v, out_v, sem):
    d = x_hbm.shape[-1]
    wid = lax.axis_index("s") * 2 + lax.axis_index("c")

    pltpu.sync_copy(x_hbm.at[wid], row_v)

    def lane_reduce(vec, op):
        s = vec[0]
        for i in range(1, _L):
            s = op(s, vec[i])
        return jnp.full((_L,), s, jnp.float32)

    def recip(x):
        seed = jnp.int32(0x7EF311C7) - lax.bitcast_convert_type(x, jnp.int32)
        r = lax.bitcast_convert_type(seed, jnp.float32)
        for _ in range(3):
            r = r * (2.0 - x * r)
        return r

    add = lambda a, b: a + b
    mvec = _sc_reduce_pass(row_v, d, jnp.maximum, jnp.maximum, -jnp.inf)
    zmax = lane_reduce(mvec, jnp.maximum)

    def relu_sum(t):
        acc = _sc_reduce_pass(
            row_v, d, lambda a, v: a + jnp.maximum(v - t, 0.0), add, 0.0)
        return lane_reduce(acc, add)

    lo = zmax - 1.0
    hi = zmax
    f_lo = relu_sum(lo) - 1.0
    t_p = lo - 1.0
    f_p = f_lo + 1.0

    def step(_, carry):
        lo, hi, f_lo, t_p, f_p = carry
        mid = 0.5 * (lo + hi)
        sec = lo + f_lo * (lo - t_p) * recip(jnp.maximum(f_p - f_lo, 1e-20))
        t = jnp.where(sec < hi, jnp.maximum(sec, mid), mid)
        ft = relu_sum(t) - 1.0
        ge = ft >= 0.0
        return (
            jnp.where(ge, t, lo),
            jnp.where(ge, hi, t),
            jnp.where(ge, ft, f_lo),
            jnp.where(ge, lo, t_p),
            jnp.where(ge, f_lo, f_p),
        )

    lo, hi, f_lo, t_p, f_p = lax.fori_loop(
        0, _SC_SOLVE_ITERS, step, (lo, hi, f_lo, t_p, f_p))

    cnt = _sc_reduce_pass(
        row_v, d,
        lambda a, v: a + jnp.where(v > lo, 1.0, 0.0), add, 0.0)
    cnt = lane_reduce(cnt, add)
    tau = lo + f_lo * recip(jnp.maximum(cnt, 1.0))

    span = _L * _UNROLL

    def out_body(i, carry):
        base = i * span
        for j in range(_UNROLL):
            sl = pl.ds(base + j * _L, _L)
            out_v[sl] = jnp.maximum(row_v[sl] - tau, 0.0)
        return carry

    lax.fori_loop(0, d // span, out_body, 0)
    pltpu.sync_copy(out_v, o_hbm.at[wid])


def _sc_sparsemax(x):
    n_rows, d = x.shape
    mesh = plsc.VectorSubcoreMesh(core_axis_name="c", subcore_axis_name="s")
    k = functools.partial(
        pl.kernel,
        mesh=mesh,
        out_type=jax.ShapeDtypeStruct((n_rows, d), jnp.float32),
        scratch_types=[
            pltpu.VMEM((d,), jnp.float32),
            pltpu.VMEM((d,), jnp.float32),
            pltpu.SemaphoreType.DMA,
        ],
    )(_sc_sparsemax_body)
    return k(x)


@jax.jit
def kernel(input):
    n_rows, _ = input.shape
    n_tc = n_rows - _SC_ROWS
    out_tc = _tc_sparsemax(input[:n_tc])
    out_sc = _sc_sparsemax(input[n_tc:])
    return jnp.concatenate([out_tc, out_sc], axis=0)


# speedup vs baseline: 1.0113x; 1.0050x over previous
import functools

import jax
import jax.numpy as jnp
from jax import lax
from jax.experimental import pallas as pl
from jax.experimental.pallas import tpu as pltpu
from jax.experimental.pallas import tpu_sc as plsc

_SOLVE_ITERS = 8
_ROW_BLOCK = 32

_SC_ROWS = 32
_SC_SOLVE_ITERS = 6
_L = 16
_UNROLL = 32


def _sparsemax_block(x_ref, o_ref):
    zmax = jnp.max(x_ref[...], axis=-1, keepdims=True)
    lo = zmax - 1.0
    hi = zmax
    f_lo = jnp.sum(jnp.maximum(x_ref[...] - lo, 0.0), axis=-1,
                   keepdims=True) - 1.0
    t_p = lo - 1.0
    f_p = f_lo + 1.0

    def step(_, carry):
        lo, hi, f_lo, t_p, f_p = carry
        mid = 0.5 * (lo + hi)
        sec = lo + f_lo * (lo - t_p) / jnp.maximum(f_p - f_lo, 1e-30)
        t = jnp.where(sec < hi, jnp.maximum(sec, mid), mid)
        ft = jnp.sum(jnp.maximum(x_ref[...] - t, 0.0), axis=-1,
                     keepdims=True) - 1.0
        ge = ft >= 0.0
        return (
            jnp.where(ge, t, lo),
            jnp.where(ge, hi, t),
            jnp.where(ge, ft, f_lo),
            jnp.where(ge, lo, t_p),
            jnp.where(ge, f_lo, f_p),
        )

    lo, hi, f_lo, t_p, f_p = jax.lax.fori_loop(
        0, _SOLVE_ITERS, step, (lo, hi, f_lo, t_p, f_p))

    cnt = jnp.sum((x_ref[...] > lo).astype(jnp.float32), axis=-1,
                  keepdims=True)
    tau = lo + f_lo / jnp.maximum(cnt, 1.0)
    o_ref[...] = jnp.maximum(x_ref[...] - tau, 0.0)


def _tc_sparsemax(x):
    n_rows, d = x.shape
    grid = (n_rows // _ROW_BLOCK,)
    return pl.pallas_call(
        _sparsemax_block,
        grid=grid,
        in_specs=[pl.BlockSpec((_ROW_BLOCK, d), lambda i: (i, 0))],
        out_specs=pl.BlockSpec((_ROW_BLOCK, d), lambda i: (i, 0)),
        out_shape=jax.ShapeDtypeStruct((n_rows, d), x.dtype),
        compiler_params=pltpu.CompilerParams(
            dimension_semantics=("parallel",),
        ),
    )(x)


def _sc_reduce_pass(row_v, d, acc_op, merge_op, init):
    span = _L * _UNROLL

    def body(i, accs):
        base = i * span
        new = []
        for j in range(_UNROLL):
            v = row_v[pl.ds(base + j * _L, _L)]
            new.append(acc_op(accs[j], v))
        return tuple(new)

    accs = tuple(jnp.full((_L,), init, jnp.float32) for _ in range(_UNROLL))
    accs = lax.fori_loop(0, d // span, body, accs)
    out = accs[0]
    for j in range(1, _UNROLL):
        out = merge_op(out, accs[j])
    return out


def _sc_sparsemax_body(x_hbm, o_hbm, row_v, out_v, sem):
    d = x_hbm.shape[-1]
    wid = lax.axis_index("s") * 2 + lax.axis_index("c")

    pltpu.sync_copy(x_hbm.at[wid], row_v)

    def lane_reduce(vec, op):
        s = vec[0]
        for i in range(1, _L):
            s = op(s, vec[i])
        return jnp.full((_L,), s, jnp.float32)

    def recip(x):
        seed = jnp.int32(0x7EF311C7) - lax.bitcast_convert_type(x, jnp.int32)
        r = lax.bitcast_convert_type(seed, jnp.float32)
        for _ in range(3):
            r = r * (2.0 - x * r)
        return r

    add = lambda a, b: a + b
    mvec = _sc_reduce_pass(row_v, d, jnp.maximum, jnp.maximum, -jnp.inf)
    zmax = lane_reduce(mvec, jnp.maximum)

    def relu_sum(t):
        acc = _sc_reduce_pass(
            row_v, d, lambda a, v: a + jnp.maximum(v - t, 0.0), add, 0.0)
        return lane_reduce(acc, add)

    lo = zmax - 1.0
    hi = zmax
    f_lo = relu_sum(lo) - 1.0
    t_p = lo - 1.0
    f_p = f_lo + 1.0

    def step(_, carry):
        lo, hi, f_lo, t_p, f_p = carry
        mid = 0.5 * (lo + hi)
        sec = lo + f_lo * (lo - t_p) * recip(jnp.maximum(f_p - f_lo, 1e-20))
        t = jnp.where(sec < hi, jnp.maximum(sec, mid), mid)
        ft = relu_sum(t) - 1.0
        ge = ft >= 0.0
        return (
            jnp.where(ge, t, lo),
            jnp.where(ge, hi, t),
            jnp.where(ge, ft, f_lo),
            jnp.where(ge, lo, t_p),
            jnp.where(ge, f_lo, f_p),
        )

    lo, hi, f_lo, t_p, f_p = lax.fori_loop(
        0, _SC_SOLVE_ITERS, step, (lo, hi, f_lo, t_p, f_p))

    cnt = _sc_reduce_pass(
        row_v, d,
        lambda a, v: a + jnp.where(v > lo, 1.0, 0.0), add, 0.0)
    cnt = lane_reduce(cnt, add)
    tau = lo + f_lo * recip(jnp.maximum(cnt, 1.0))

    span = _L * _UNROLL

    def out_body(i, carry):
        base = i * span
        for j in range(_UNROLL):
            sl = pl.ds(base + j * _L, _L)
            out_v[sl] = jnp.maximum(row_v[sl] - tau, 0.0)
        return carry

    lax.fori_loop(0, d // span, out_body, 0)
    pltpu.sync_copy(out_v, o_hbm.at[wid])


def _sc_sparsemax(x):
    n_rows, d = x.shape
    mesh = plsc.VectorSubcoreMesh(core_axis_name="c", subcore_axis_name="s")
    k = functools.partial(
        pl.kernel,
        mesh=mesh,
        out_type=jax.ShapeDtypeStruct((n_rows, d), jnp.float32),
        scratch_types=[
            pltpu.VMEM((d,), jnp.float32),
            pltpu.VMEM((d,), jnp.float32),
            pltpu.SemaphoreType.DMA,
        ],
    )(_sc_sparsemax_body)
    return k(x)


@jax.jit
def kernel(input):
    n_rows, _ = input.shape
    n_tc = n_rows - _SC_ROWS
    out_sc = _sc_sparsemax(input[n_tc:])
    out_tc = _tc_sparsemax(input[:n_tc])
    return jnp.concatenate([out_tc, out_sc], axis=0)


# speedup vs baseline: 2.2458x; 2.2207x over previous
import jax
import jax.numpy as jnp
from jax.experimental import pallas as pl
from jax.experimental.pallas import tpu as pltpu

_SOLVE_ITERS = 8
_ROW_BLOCK = 64


def _sparsemax_block(x_ref, o_ref):
    zmax = jnp.max(x_ref[...], axis=-1, keepdims=True)
    lo = zmax - 1.0
    hi = zmax
    f_lo = jnp.sum(jnp.maximum(x_ref[...] - lo, 0.0), axis=-1,
                   keepdims=True) - 1.0
    t_p = lo - 1.0
    f_p = f_lo + 1.0

    def step(_, carry):
        lo, hi, f_lo, t_p, f_p = carry
        mid = 0.5 * (lo + hi)
        sec = lo + f_lo * (lo - t_p) / jnp.maximum(f_p - f_lo, 1e-30)
        t = jnp.where(sec < hi, jnp.maximum(sec, mid), mid)
        ft = jnp.sum(jnp.maximum(x_ref[...] - t, 0.0), axis=-1,
                     keepdims=True) - 1.0
        ge = ft >= 0.0
        return (
            jnp.where(ge, t, lo),
            jnp.where(ge, hi, t),
            jnp.where(ge, ft, f_lo),
            jnp.where(ge, lo, t_p),
            jnp.where(ge, f_lo, f_p),
        )

    lo, hi, f_lo, t_p, f_p = jax.lax.fori_loop(
        0, _SOLVE_ITERS, step, (lo, hi, f_lo, t_p, f_p))

    cnt = jnp.sum((x_ref[...] > lo).astype(jnp.float32), axis=-1,
                  keepdims=True)
    tau = lo + f_lo / jnp.maximum(cnt, 1.0)
    o_ref[...] = jnp.maximum(x_ref[...] - tau, 0.0)


@jax.jit
def kernel(input):
    n_rows, d = input.shape
    grid = (n_rows // _ROW_BLOCK,)
    return pl.pallas_call(
        _sparsemax_block,
        grid=grid,
        in_specs=[pl.BlockSpec((_ROW_BLOCK, d), lambda i: (i, 0))],
        out_specs=pl.BlockSpec((_ROW_BLOCK, d), lambda i: (i, 0)),
        out_shape=jax.ShapeDtypeStruct((n_rows, d), input.dtype),
        compiler_params=pltpu.CompilerParams(
            dimension_semantics=("parallel",),
        ),
    )(input)
